# Initial kernel scaffold; baseline (speedup 1.0000x reference)
import jax
import jax.numpy as jnp
from jax.experimental import pallas as pl


def kernel(x, edge_index, W_rel, W_msg, W_self, W_upd, W_dec):
    raise NotImplementedError("write your pallas kernel here")


# speedup vs baseline: 4.5148x; 4.5148x over previous
import functools

import jax
import jax.numpy as jnp
from jax import lax
from jax.experimental import pallas as pl
from jax.experimental.pallas import tpu as pltpu
from jax.experimental.pallas import tpu_sc as plsc

N = 10000
E = 320000
D = 128
DH = 64
AB = 16
KT5 = 900

NC = 2
NS = 16
NW = NC * NS
EPW = E // NW
B = 400
CH = EPW // B
RSTEP = 624
RPT = 640
BR = 1000


def _pre_body(x_ref, wmsg_ref, wself_ref, wr2_ref, y0_ref, y1_ref, h0_ref,
              ab_ref):
    xb = x_ref[...]
    ym = jnp.dot(xb, wmsg_ref[...], preferred_element_type=jnp.float32)
    y0_ref[...] = ym[:, :DH]
    y1_ref[...] = ym[:, DH:]
    h0_ref[...] = jnp.dot(xb, wself_ref[...], preferred_element_type=jnp.float32)
    ab_ref[...] = jnp.dot(xb, wr2_ref[...], preferred_element_type=jnp.float32)


def _tail_body(h0_ref, p00_ref, p01_ref, p10_ref, p11_ref, wu0_ref, wu1_ref,
               wdec_ref, out_ref):
    agg0 = p00_ref[...] + p10_ref[...]
    agg1 = p01_ref[...] + p11_ref[...]
    h = (h0_ref[...]
         + jnp.dot(agg0, wu0_ref[...], preferred_element_type=jnp.float32)
         + jnp.dot(agg1, wu1_ref[...], preferred_element_type=jnp.float32))
    h = jnp.maximum(h, 0.0)
    out_ref[...] = jnp.dot(h, wdec_ref[...], preferred_element_type=jnp.float32)


def _sc_body(src_hbm, dst_hbm, y0_hbm, y1_hbm, ab_hbm, zeros_hbm, out_hbm,
             agg_sh, src_v, dst_v, abs_v, abd_v, rows_v, gates_v,
             sem_rows, sem_ab):
    cid = lax.axis_index("c")
    sid = lax.axis_index("s")
    wid = cid * NS + sid
    base_n = sid * RSTEP

    for half in range(2):
        yh_hbm = y0_hbm if half == 0 else y1_hbm
        pltpu.sync_copy(zeros_hbm.at[pl.ds(base_n, RPT)],
                        agg_sh.at[pl.ds(base_n, RPT)])
        plsc.subcore_barrier()

        def chunk(t, carry):
            ebase = wid * EPW + t * B
            pltpu.sync_copy(src_hbm.at[pl.ds(ebase, B)], src_v)
            pltpu.sync_copy(dst_hbm.at[pl.ds(ebase, B)], dst_v)
            gather = pltpu.async_copy(yh_hbm.at[src_v], rows_v, sem_rows)
            ga = pltpu.async_copy(ab_hbm.at[src_v], abs_v, sem_ab)
            gb = pltpu.async_copy(ab_hbm.at[dst_v], abd_v, sem_ab)
            ga.wait()
            gb.wait()
            for g in range(B // 16):
                ev = jnp.arange(16, dtype=jnp.int32) + (g * 16)
                l = []
                for k in range(4):
                    a_k = plsc.load_gather(
                        abs_v, [ev, jnp.full((16,), k, jnp.int32)])
                    b_k = plsc.load_gather(
                        abd_v, [ev, jnp.full((16,), 4 + k, jnp.int32)])
                    l.append(a_k + b_k)
                m = jnp.maximum(jnp.maximum(l[0], l[1]),
                                jnp.maximum(l[2], l[3]))
                e0 = jnp.exp(l[0] - m)
                s123 = (jnp.exp(l[1] - m) + jnp.exp(l[2] - m)
                        + jnp.exp(l[3] - m))
                gates_v[pl.ds(g * 16, 16)] = s123 / (e0 + s123)
            gather.wait()

            def scale_row(e2, c2):
                gsc = plsc.load_gather(
                    gates_v, [jnp.full((16,), 0, jnp.int32) + e2])
                for j in range(DH // 16):
                    rows_v[e2, pl.ds(j * 16, 16)] = (
                        rows_v[e2, pl.ds(j * 16, 16)] * gsc)
                return c2
            lax.fori_loop(0, B, scale_row, 0)

            pltpu.sync_copy(rows_v, agg_sh.at[dst_v], add=True)
            return carry

        lax.fori_loop(0, CH, chunk, 0)
        plsc.subcore_barrier()
        pltpu.sync_copy(agg_sh.at[pl.ds(base_n, RPT)],
                        out_hbm.at[cid, half, pl.ds(base_n, RPT)])
        plsc.subcore_barrier()


@functools.cache
def _sc_agg():
    return pl.kernel(
        _sc_body,
        out_type=jax.ShapeDtypeStruct((NC, 2, N, DH), jnp.float32),
        mesh=plsc.VectorSubcoreMesh(core_axis_name="c", subcore_axis_name="s",
                                    num_cores=NC, num_subcores=NS),
        scratch_types=[
            pltpu.VMEM_SHARED((N, DH), jnp.float32),
            pltpu.VMEM((B,), jnp.int32),
            pltpu.VMEM((B,), jnp.int32),
            pltpu.VMEM((B, AB), jnp.float32),
            pltpu.VMEM((B, AB), jnp.float32),
            pltpu.VMEM((B, DH), jnp.float32),
            pltpu.VMEM((B,), jnp.float32),
            pltpu.SemaphoreType.DMA,
            pltpu.SemaphoreType.DMA,
        ],
        compiler_params=pltpu.CompilerParams(needs_layout_passes=False,
                                             use_tc_tiling_on_sc=False),
    )


@jax.jit
def kernel(x, edge_index, W_rel, W_msg, W_self, W_upd, W_dec):
    wr2 = jnp.concatenate(
        [W_rel[:D], W_rel[D:], jnp.zeros((D, AB - 8), jnp.float32)], axis=1)

    y0, y1, h0, ab = pl.pallas_call(
        _pre_body,
        grid=(N // BR,),
        in_specs=[
            pl.BlockSpec((BR, D), lambda i: (i, 0)),
            pl.BlockSpec((D, D), lambda i: (0, 0)),
            pl.BlockSpec((D, D), lambda i: (0, 0)),
            pl.BlockSpec((D, AB), lambda i: (0, 0)),
        ],
        out_specs=[
            pl.BlockSpec((BR, DH), lambda i: (i, 0)),
            pl.BlockSpec((BR, DH), lambda i: (i, 0)),
            pl.BlockSpec((BR, D), lambda i: (i, 0)),
            pl.BlockSpec((BR, AB), lambda i: (i, 0)),
        ],
        out_shape=[
            jax.ShapeDtypeStruct((N, DH), jnp.float32),
            jax.ShapeDtypeStruct((N, DH), jnp.float32),
            jax.ShapeDtypeStruct((N, D), jnp.float32),
            jax.ShapeDtypeStruct((N, AB), jnp.float32),
        ],
    )(x, W_msg, W_self, wr2)

    src = edge_index[0]
    dst = edge_index[1]
    zeros = jnp.zeros((N, DH), jnp.float32)
    parts = _sc_agg()(src, dst, y0, y1, ab, zeros)

    out = pl.pallas_call(
        _tail_body,
        grid=(N // BR,),
        in_specs=[
            pl.BlockSpec((BR, D), lambda i: (i, 0)),
            pl.BlockSpec((BR, DH), lambda i: (i, 0)),
            pl.BlockSpec((BR, DH), lambda i: (i, 0)),
            pl.BlockSpec((BR, DH), lambda i: (i, 0)),
            pl.BlockSpec((BR, DH), lambda i: (i, 0)),
            pl.BlockSpec((DH, D), lambda i: (0, 0)),
            pl.BlockSpec((DH, D), lambda i: (1, 0)),
            pl.BlockSpec((D, KT5), lambda i: (0, 0)),
        ],
        out_specs=pl.BlockSpec((BR, KT5), lambda i: (i, 0)),
        out_shape=jax.ShapeDtypeStruct((N, KT5), jnp.float32),
    )(h0, parts[0, 0], parts[0, 1], parts[1, 0], parts[1, 1],
      W_upd, W_upd, W_dec)

    return out.reshape(N, 6, 30, 5)


# speedup vs baseline: 6.5261x; 1.4455x over previous
import functools

import jax
import jax.numpy as jnp
from jax import lax
from jax.experimental import pallas as pl
from jax.experimental.pallas import tpu as pltpu
from jax.experimental.pallas import tpu_sc as plsc

N = 10000
E = 320000
D = 128
DH = 64
AB = 16
KT5 = 900

NC = 2
NS = 16
NW = NC * NS
EPW = E // NW
B = 80
CH = EPW // B
RSTEP = 624
RPT = 640
BR = 1000


def _pre_body(x_ref, wmsg_ref, wself_ref, wr2_ref, y0_ref, y1_ref, h0_ref,
              ab_ref):
    xb = x_ref[...]
    ym = jnp.dot(xb, wmsg_ref[...], preferred_element_type=jnp.float32)
    y0_ref[...] = ym[:, :DH]
    y1_ref[...] = ym[:, DH:]
    h0_ref[...] = jnp.dot(xb, wself_ref[...], preferred_element_type=jnp.float32)
    ab_ref[...] = jnp.dot(xb, wr2_ref[...], preferred_element_type=jnp.float32)


def _tail_body(h0_ref, p00_ref, p01_ref, p10_ref, p11_ref, wu0_ref, wu1_ref,
               wdec_ref, out_ref):
    agg0 = p00_ref[...] + p10_ref[...]
    agg1 = p01_ref[...] + p11_ref[...]
    h = (h0_ref[...]
         + jnp.dot(agg0, wu0_ref[...], preferred_element_type=jnp.float32)
         + jnp.dot(agg1, wu1_ref[...], preferred_element_type=jnp.float32))
    h = jnp.maximum(h, 0.0)
    out_ref[...] = jnp.dot(h, wdec_ref[...], preferred_element_type=jnp.float32)


def _sc_body(ei4_hbm, y0_hbm, y1_hbm, ab_hbm, zeros_hbm, out_hbm,
             agg_sh, srcs_v, dsts_v, abs0, abd0, rows0, abs1, abd1, rows1,
             gates_v, sem0, sem1):
    cid = lax.axis_index("c")
    sid = lax.axis_index("s")
    wid = cid * NS + sid
    base_n = sid * RSTEP

    pltpu.sync_copy(ei4_hbm.at[0, wid], srcs_v)
    pltpu.sync_copy(ei4_hbm.at[1, wid], dsts_v)

    abs_b, abd_b, rows_b, sem_b = (abs0, abs1), (abd0, abd1), \
        (rows0, rows1), (sem0, sem1)

    for half in range(2):
        yh_hbm = y0_hbm if half == 0 else y1_hbm
        pltpu.sync_copy(zeros_hbm.at[pl.ds(base_n, RPT)],
                        agg_sh.at[pl.ds(base_n, RPT)])
        plsc.subcore_barrier()

        def issue(t, b):
            si = srcs_v.at[t]
            pltpu.async_copy(ab_hbm.at[si], abs_b[b], sem_b[b])
            pltpu.async_copy(ab_hbm.at[dsts_v.at[t]], abd_b[b], sem_b[b])
            pltpu.async_copy(yh_hbm.at[si], rows_b[b], sem_b[b])

        def process(t, b):
            pltpu.make_async_copy(ab_hbm.at[srcs_v.at[t]], abs_b[b],
                                  sem_b[b]).wait()
            pltpu.make_async_copy(ab_hbm.at[dsts_v.at[t]], abd_b[b],
                                  sem_b[b]).wait()
            pltpu.make_async_copy(yh_hbm.at[srcs_v.at[t]], rows_b[b],
                                  sem_b[b]).wait()
            for g in range(B // 16):
                ev = jnp.arange(16, dtype=jnp.int32) + (g * 16)
                l = []
                for k in range(4):
                    a_k = plsc.load_gather(
                        abs_b[b], [ev, jnp.full((16,), k, jnp.int32)])
                    b_k = plsc.load_gather(
                        abd_b[b], [ev, jnp.full((16,), 4 + k, jnp.int32)])
                    l.append(a_k + b_k)
                m = jnp.maximum(jnp.maximum(l[0], l[1]),
                                jnp.maximum(l[2], l[3]))
                e0 = jnp.exp(l[0] - m)
                s123 = (jnp.exp(l[1] - m) + jnp.exp(l[2] - m)
                        + jnp.exp(l[3] - m))
                gates_v[pl.ds(g * 16, 16)] = s123 / (e0 + s123)

            rv = rows_b[b]

            @plsc.parallel_loop(0, B, 1, unroll=4)
            def scale_row(e2):
                gsc = plsc.load_gather(
                    gates_v, [jnp.full((16,), 0, jnp.int32) + e2])
                for j in range(DH // 16):
                    rv[e2, pl.ds(j * 16, 16)] = rv[e2, pl.ds(j * 16, 16)] * gsc

            pltpu.sync_copy(rows_b[b], agg_sh.at[dsts_v.at[t]], add=True)

        issue(0, 0)

        def pair(p, carry):
            t1 = 2 * p + 1
            issue(t1, 1)
            process(2 * p, 0)
            issue(t1 + 1, 0)
            process(t1, 1)
            return carry

        lax.fori_loop(0, (CH - 1) // 2, pair, 0)
        process(CH - 1, 0)
        plsc.subcore_barrier()
        pltpu.sync_copy(agg_sh.at[pl.ds(base_n, RPT)],
                        out_hbm.at[cid, half, pl.ds(base_n, RPT)])
        plsc.subcore_barrier()


@functools.cache
def _sc_agg():
    return pl.kernel(
        _sc_body,
        out_type=jax.ShapeDtypeStruct((NC, 2, N, DH), jnp.float32),
        mesh=plsc.VectorSubcoreMesh(core_axis_name="c", subcore_axis_name="s",
                                    num_cores=NC, num_subcores=NS),
        scratch_types=[
            pltpu.VMEM_SHARED((N, DH), jnp.float32),
            pltpu.VMEM((CH, B), jnp.int32),
            pltpu.VMEM((CH, B), jnp.int32),
            pltpu.VMEM((B, AB), jnp.float32),
            pltpu.VMEM((B, AB), jnp.float32),
            pltpu.VMEM((B, DH), jnp.float32),
            pltpu.VMEM((B, AB), jnp.float32),
            pltpu.VMEM((B, AB), jnp.float32),
            pltpu.VMEM((B, DH), jnp.float32),
            pltpu.VMEM((B,), jnp.float32),
            pltpu.SemaphoreType.DMA,
            pltpu.SemaphoreType.DMA,
        ],
        compiler_params=pltpu.CompilerParams(needs_layout_passes=False,
                                             use_tc_tiling_on_sc=False),
    )


@jax.jit
def kernel(x, edge_index, W_rel, W_msg, W_self, W_upd, W_dec):
    wr2 = jnp.concatenate(
        [W_rel[:D], W_rel[D:], jnp.zeros((D, AB - 8), jnp.float32)], axis=1)

    y0, y1, h0, ab = pl.pallas_call(
        _pre_body,
        grid=(N // BR,),
        in_specs=[
            pl.BlockSpec((BR, D), lambda i: (i, 0)),
            pl.BlockSpec((D, D), lambda i: (0, 0)),
            pl.BlockSpec((D, D), lambda i: (0, 0)),
            pl.BlockSpec((D, AB), lambda i: (0, 0)),
        ],
        out_specs=[
            pl.BlockSpec((BR, DH), lambda i: (i, 0)),
            pl.BlockSpec((BR, DH), lambda i: (i, 0)),
            pl.BlockSpec((BR, D), lambda i: (i, 0)),
            pl.BlockSpec((BR, AB), lambda i: (i, 0)),
        ],
        out_shape=[
            jax.ShapeDtypeStruct((N, DH), jnp.float32),
            jax.ShapeDtypeStruct((N, DH), jnp.float32),
            jax.ShapeDtypeStruct((N, D), jnp.float32),
            jax.ShapeDtypeStruct((N, AB), jnp.float32),
        ],
    )(x, W_msg, W_self, wr2)

    ei4 = edge_index.reshape(2, NW, CH, B)
    zeros = jnp.zeros((N, DH), jnp.float32)
    parts = _sc_agg()(ei4, y0, y1, ab, zeros)

    out = pl.pallas_call(
        _tail_body,
        grid=(N // BR,),
        in_specs=[
            pl.BlockSpec((BR, D), lambda i: (i, 0)),
            pl.BlockSpec((BR, DH), lambda i: (i, 0)),
            pl.BlockSpec((BR, DH), lambda i: (i, 0)),
            pl.BlockSpec((BR, DH), lambda i: (i, 0)),
            pl.BlockSpec((BR, DH), lambda i: (i, 0)),
            pl.BlockSpec((DH, D), lambda i: (0, 0)),
            pl.BlockSpec((DH, D), lambda i: (1, 0)),
            pl.BlockSpec((D, KT5), lambda i: (0, 0)),
        ],
        out_specs=pl.BlockSpec((BR, KT5), lambda i: (i, 0)),
        out_shape=jax.ShapeDtypeStruct((N, KT5), jnp.float32),
    )(h0, parts[0, 0], parts[0, 1], parts[1, 0], parts[1, 1],
      W_upd, W_upd, W_dec)

    return out.reshape(N, 6, 30, 5)


# speedup vs baseline: 6.6365x; 1.0169x over previous
import functools

import jax
import jax.numpy as jnp
from jax import lax
from jax.experimental import pallas as pl
from jax.experimental.pallas import tpu as pltpu
from jax.experimental.pallas import tpu_sc as plsc

N = 10000
E = 320000
D = 128
DH = 64
AB = 16
KT5 = 900

NC = 2
NS = 16
NW = NC * NS
EPW = E // NW
B = 80
CH = EPW // B
RSTEP = 624
RPT = 640
BR = 1000


def _pre_body(x_ref, wmsg_ref, wself_ref, wr2_ref, y0_ref, y1_ref, h0_ref,
              ab_ref, z_ref):
    xb = x_ref[...]
    ym = jnp.dot(xb, wmsg_ref[...], preferred_element_type=jnp.float32)
    y0_ref[...] = ym[:, :DH]
    y1_ref[...] = ym[:, DH:]
    h0_ref[...] = jnp.dot(xb, wself_ref[...], preferred_element_type=jnp.float32)
    ab_ref[...] = jnp.dot(xb, wr2_ref[...], preferred_element_type=jnp.float32)
    z_ref[...] = jnp.zeros((BR, DH), jnp.float32)


def _tail_body(h0_ref, p00_ref, p01_ref, p10_ref, p11_ref, wu0_ref, wu1_ref,
               wdec_ref, out_ref):
    agg0 = p00_ref[0, 0] + p10_ref[0, 0]
    agg1 = p01_ref[0, 0] + p11_ref[0, 0]
    h = (h0_ref[...]
         + jnp.dot(agg0, wu0_ref[...], preferred_element_type=jnp.float32)
         + jnp.dot(agg1, wu1_ref[...], preferred_element_type=jnp.float32))
    h = jnp.maximum(h, 0.0)
    out_ref[...] = jnp.dot(h, wdec_ref[...], preferred_element_type=jnp.float32)


def _sc_body(ei_hbm, y0_hbm, y1_hbm, ab_hbm, zeros_hbm, out_hbm,
             agg_sh, srcs_v, dsts_v, abs0, abd0, rows0, abs1, abd1, rows1,
             gates_v, sem0, sem1, sems0, sems1, semi):
    cid = lax.axis_index("c")
    sid = lax.axis_index("s")
    wid = cid * NS + sid
    base_n = sid * RSTEP

    def _pre_issue(t, c):
        off = wid * EPW + t * B
        pltpu.async_copy(ei_hbm.at[0, pl.ds(off, B)], srcs_v.at[t], semi)
        pltpu.async_copy(ei_hbm.at[1, pl.ds(off, B)], dsts_v.at[t], semi)
        return c
    lax.fori_loop(0, CH, _pre_issue, 0)

    def _pre_drain(t, c):
        off = wid * EPW + t * B
        pltpu.make_async_copy(ei_hbm.at[0, pl.ds(off, B)], srcs_v.at[t],
                              semi).wait()
        pltpu.make_async_copy(ei_hbm.at[1, pl.ds(off, B)], dsts_v.at[t],
                              semi).wait()
        return c
    lax.fori_loop(0, CH, _pre_drain, 0)

    abs_b, abd_b, rows_b, sem_b = (abs0, abs1), (abd0, abd1), \
        (rows0, rows1), (sem0, sem1)
    sems_b = (sems0, sems1)

    for half in range(2):
        yh_hbm = y0_hbm if half == 0 else y1_hbm
        pltpu.sync_copy(zeros_hbm.at[pl.ds(base_n, RPT)],
                        agg_sh.at[pl.ds(base_n, RPT)])
        plsc.subcore_barrier()

        def issue(t, b):
            @pl.when(t >= 2)
            def _():
                pltpu.make_async_copy(rows_b[b], agg_sh.at[dsts_v.at[t]],
                                      sems_b[b]).wait()
            si = srcs_v.at[t]
            pltpu.async_copy(ab_hbm.at[si], abs_b[b], sem_b[b])
            pltpu.async_copy(ab_hbm.at[dsts_v.at[t]], abd_b[b], sem_b[b])
            pltpu.async_copy(yh_hbm.at[si], rows_b[b], sem_b[b])

        def process(t, b):
            pltpu.make_async_copy(ab_hbm.at[srcs_v.at[t]], abs_b[b],
                                  sem_b[b]).wait()
            pltpu.make_async_copy(ab_hbm.at[dsts_v.at[t]], abd_b[b],
                                  sem_b[b]).wait()
            pltpu.make_async_copy(yh_hbm.at[srcs_v.at[t]], rows_b[b],
                                  sem_b[b]).wait()
            for g in range(B // 16):
                ev = jnp.arange(16, dtype=jnp.int32) + (g * 16)
                l = []
                for k in range(4):
                    a_k = plsc.load_gather(
                        abs_b[b], [ev, jnp.full((16,), k, jnp.int32)])
                    b_k = plsc.load_gather(
                        abd_b[b], [ev, jnp.full((16,), 4 + k, jnp.int32)])
                    l.append(a_k + b_k)
                m = jnp.maximum(jnp.maximum(l[0], l[1]),
                                jnp.maximum(l[2], l[3]))
                e0 = jnp.exp(l[0] - m)
                s123 = (jnp.exp(l[1] - m) + jnp.exp(l[2] - m)
                        + jnp.exp(l[3] - m))
                gates_v[pl.ds(g * 16, 16)] = s123 / (e0 + s123)

            rv = rows_b[b]

            @plsc.parallel_loop(0, B, 1, unroll=8)
            def scale_row(e2):
                gsc = plsc.load_gather(
                    gates_v, [jnp.full((16,), 0, jnp.int32) + e2])
                for j in range(DH // 16):
                    rv[e2, pl.ds(j * 16, 16)] = rv[e2, pl.ds(j * 16, 16)] * gsc

            pltpu.make_async_copy(rows_b[b], agg_sh.at[dsts_v.at[t]],
                                  sems_b[b]).start(add=True)

        issue(0, 0)

        def pair(p, carry):
            t1 = 2 * p + 1
            issue(t1, 1)
            process(2 * p, 0)
            issue(t1 + 1, 0)
            process(t1, 1)
            return carry

        lax.fori_loop(0, (CH - 1) // 2, pair, 0)
        process(CH - 1, 0)
        pltpu.make_async_copy(rows_b[0], agg_sh.at[dsts_v.at[CH - 1]],
                              sems_b[0]).wait()
        pltpu.make_async_copy(rows_b[1], agg_sh.at[dsts_v.at[CH - 2]],
                              sems_b[1]).wait()
        plsc.subcore_barrier()
        pltpu.sync_copy(agg_sh.at[pl.ds(base_n, RPT)],
                        out_hbm.at[cid, half, pl.ds(base_n, RPT)])
        plsc.subcore_barrier()


@functools.cache
def _sc_agg():
    return pl.kernel(
        _sc_body,
        out_type=jax.ShapeDtypeStruct((NC, 2, N, DH), jnp.float32),
        mesh=plsc.VectorSubcoreMesh(core_axis_name="c", subcore_axis_name="s",
                                    num_cores=NC, num_subcores=NS),
        scratch_types=[
            pltpu.VMEM_SHARED((N, DH), jnp.float32),
            pltpu.VMEM((CH, B), jnp.int32),
            pltpu.VMEM((CH, B), jnp.int32),
            pltpu.VMEM((B, AB), jnp.float32),
            pltpu.VMEM((B, AB), jnp.float32),
            pltpu.VMEM((B, DH), jnp.float32),
            pltpu.VMEM((B, AB), jnp.float32),
            pltpu.VMEM((B, AB), jnp.float32),
            pltpu.VMEM((B, DH), jnp.float32),
            pltpu.VMEM((B,), jnp.float32),
            pltpu.SemaphoreType.DMA,
            pltpu.SemaphoreType.DMA,
            pltpu.SemaphoreType.DMA,
            pltpu.SemaphoreType.DMA,
            pltpu.SemaphoreType.DMA,
        ],
        compiler_params=pltpu.CompilerParams(needs_layout_passes=False,
                                             use_tc_tiling_on_sc=False),
    )


@jax.jit
def kernel(x, edge_index, W_rel, W_msg, W_self, W_upd, W_dec):
    wr2 = jnp.concatenate(
        [W_rel[:D], W_rel[D:], jnp.zeros((D, AB - 8), jnp.float32)], axis=1)

    y0, y1, h0, ab, zeros = pl.pallas_call(
        _pre_body,
        grid=(N // BR,),
        in_specs=[
            pl.BlockSpec((BR, D), lambda i: (i, 0)),
            pl.BlockSpec((D, D), lambda i: (0, 0)),
            pl.BlockSpec((D, D), lambda i: (0, 0)),
            pl.BlockSpec((D, AB), lambda i: (0, 0)),
        ],
        out_specs=[
            pl.BlockSpec((BR, DH), lambda i: (i, 0)),
            pl.BlockSpec((BR, DH), lambda i: (i, 0)),
            pl.BlockSpec((BR, D), lambda i: (i, 0)),
            pl.BlockSpec((BR, AB), lambda i: (i, 0)),
            pl.BlockSpec((BR, DH), lambda i: (i, 0)),
        ],
        out_shape=[
            jax.ShapeDtypeStruct((N, DH), jnp.float32),
            jax.ShapeDtypeStruct((N, DH), jnp.float32),
            jax.ShapeDtypeStruct((N, D), jnp.float32),
            jax.ShapeDtypeStruct((N, AB), jnp.float32),
            jax.ShapeDtypeStruct((N, DH), jnp.float32),
        ],
    )(x, W_msg, W_self, wr2)

    parts = _sc_agg()(edge_index, y0, y1, ab, zeros)

    out = pl.pallas_call(
        _tail_body,
        grid=(N // BR,),
        in_specs=[
            pl.BlockSpec((BR, D), lambda i: (i, 0)),
            pl.BlockSpec((1, 1, BR, DH), lambda i: (0, 0, i, 0)),
            pl.BlockSpec((1, 1, BR, DH), lambda i: (0, 1, i, 0)),
            pl.BlockSpec((1, 1, BR, DH), lambda i: (1, 0, i, 0)),
            pl.BlockSpec((1, 1, BR, DH), lambda i: (1, 1, i, 0)),
            pl.BlockSpec((DH, D), lambda i: (0, 0)),
            pl.BlockSpec((DH, D), lambda i: (1, 0)),
            pl.BlockSpec((D, KT5), lambda i: (0, 0)),
        ],
        out_specs=pl.BlockSpec((BR, KT5), lambda i: (i, 0)),
        out_shape=jax.ShapeDtypeStruct((N, KT5), jnp.float32),
    )(h0, parts, parts, parts, parts, W_upd, W_upd, W_dec)

    return out.reshape(N, 6, 30, 5)
